# baseline (speedup 1.0000x reference)
import functools

import jax
import jax.numpy as jnp
from jax import lax
from jax.experimental import pallas as pl
from jax.experimental.pallas import tpu as pltpu
from jax.experimental.pallas import tpu_sc as plsc

HIDDEN = 64
SCALE = float(HIDDEN) ** 0.5

NC = 2
NS = 16
NW = NC * NS
LANES = 16

CHUNK = 1024
SUB = 128
N_SUB = CHUNK // SUB


def _embed_body(token_hbm, table_hbm, out_hbm, idx_v, rows_v, sem):
    wid = lax.axis_index("s") * NC + lax.axis_index("c")
    n_rows = out_hbm.shape[0]
    per_worker = n_rows // NW
    n_chunks = per_worker // CHUNK
    base = wid * per_worker

    def chunk_body(c, carry):
        off = base + c * CHUNK
        pltpu.sync_copy(token_hbm.at[pl.ds(off, CHUNK)], idx_v)
        handles = []
        for j in range(N_SUB):
            handles.append(
                pltpu.async_copy(
                    table_hbm.at[idx_v.at[pl.ds(j * SUB, SUB)]],
                    rows_v.at[pl.ds(j * SUB, SUB)],
                    sem,
                )
            )
        for h in handles:
            h.wait()

        def scale_row(r, carry2):
            for q in range(HIDDEN // LANES):
                sl = pl.ds(q * LANES, LANES)
                rows_v[r, sl] = rows_v[r, sl] * SCALE
            return carry2

        lax.fori_loop(0, CHUNK, scale_row, 0, unroll=4)
        pltpu.sync_copy(rows_v, out_hbm.at[pl.ds(off, CHUNK)])
        return carry

    lax.fori_loop(0, n_chunks, chunk_body, 0)


def kernel(token, table):
    b, s = token.shape
    n = b * s
    flat_tok = token.reshape(n)
    mesh = plsc.VectorSubcoreMesh(core_axis_name="c", subcore_axis_name="s")
    out = pl.kernel(
        _embed_body,
        out_type=jax.ShapeDtypeStruct((n, HIDDEN), jnp.float32),
        mesh=mesh,
        scratch_types=[
            pltpu.VMEM((CHUNK,), jnp.int32),
            pltpu.VMEM((CHUNK, HIDDEN), jnp.float32),
            pltpu.SemaphoreType.DMA,
        ],
        compiler_params=pltpu.CompilerParams(use_tc_tiling_on_sc=False),
    )(flat_tok, table)
    return out.reshape(b, s, HIDDEN)


# speedup vs baseline: 1.0513x; 1.0513x over previous
import jax
import jax.numpy as jnp
from jax import lax
from jax.experimental import pallas as pl
from jax.experimental.pallas import tpu as pltpu
from jax.experimental.pallas import tpu_sc as plsc

HIDDEN = 64
SCALE = float(HIDDEN) ** 0.5

NC = 2
NS = 16
NW = NC * NS
LANES = 16

CHUNK = 640
SUB = 128
N_SUB = CHUNK // SUB


def _embed_body(token_hbm, table_hbm, out_hbm,
                idx0, idx1, rows0, rows1, gsem0, gsem1, osem0, osem1):
    wid = lax.axis_index("s") * NC + lax.axis_index("c")
    n_rows = out_hbm.shape[0]
    per_worker = n_rows // NW
    n_chunks = per_worker // CHUNK
    base = wid * per_worker

    idx = (idx0, idx1)
    rows = (rows0, rows1)
    gsem = (gsem0, gsem1)
    osem = (osem0, osem1)

    def start(c, b):
        off = base + c * CHUNK
        pltpu.sync_copy(token_hbm.at[pl.ds(off, CHUNK)], idx[b])
        for j in range(N_SUB):
            pltpu.async_copy(
                table_hbm.at[idx[b].at[pl.ds(j * SUB, SUB)]],
                rows[b].at[pl.ds(j * SUB, SUB)],
                gsem[b],
            )

    def wait_gathers(b):
        pltpu.make_async_copy(
            out_hbm.at[pl.ds(0, CHUNK)], rows[b], gsem[b]
        ).wait()

    def wait_store(b):
        pltpu.make_async_copy(
            rows[b], out_hbm.at[pl.ds(0, CHUNK)], osem[b]
        ).wait()

    def scale_rows(b):
        def scale_row(r, carry):
            for q in range(HIDDEN // LANES):
                sl = pl.ds(q * LANES, LANES)
                rows[b][r, sl] = rows[b][r, sl] * SCALE
            return carry

        lax.fori_loop(0, CHUNK, scale_row, 0, unroll=4)

    def store(c, b):
        off = base + c * CHUNK
        pltpu.async_copy(rows[b], out_hbm.at[pl.ds(off, CHUNK)], osem[b])

    start(0, 0)

    def pair_body(p, carry):
        for b in range(2):
            c = p * 2 + b
            nb = 1 - b
            @pl.when(c >= 1)
            def _():
                wait_store(nb)

            @pl.when(c + 1 < n_chunks)
            def _():
                start(c + 1, nb)

            wait_gathers(b)
            scale_rows(b)
            store(c, b)
        return carry

    lax.fori_loop(0, n_chunks // 2, pair_body, 0)
    wait_store((n_chunks - 1) % 2)


def kernel(token, table):
    b, s = token.shape
    n = b * s
    flat_tok = token.reshape(n)
    mesh = plsc.VectorSubcoreMesh(core_axis_name="c", subcore_axis_name="s")
    out = pl.kernel(
        _embed_body,
        out_type=jax.ShapeDtypeStruct((n, HIDDEN), jnp.float32),
        mesh=mesh,
        scratch_types=[
            pltpu.VMEM((CHUNK,), jnp.int32),
            pltpu.VMEM((CHUNK,), jnp.int32),
            pltpu.VMEM((CHUNK, HIDDEN), jnp.float32),
            pltpu.VMEM((CHUNK, HIDDEN), jnp.float32),
            pltpu.SemaphoreType.DMA,
            pltpu.SemaphoreType.DMA,
            pltpu.SemaphoreType.DMA,
            pltpu.SemaphoreType.DMA,
        ],
        compiler_params=pltpu.CompilerParams(use_tc_tiling_on_sc=False),
    )(flat_tok, table)
    return out.reshape(b, s, HIDDEN)
